# baseline (device time: 377288 ns/iter reference)
import jax
import jax.numpy as jnp
from jax import lax
from jax.experimental import pallas as pl
from jax.experimental.pallas import tpu as pltpu

N_DEV = 16


def _mm(a, b):
    return lax.dot_general(
        a.astype(jnp.bfloat16),
        b.astype(jnp.bfloat16),
        (((1,), (0,)), ((), ())),
        preferred_element_type=jnp.float32,
    )


def kernel(x, w_mat, scale_x, scale_w):
    m_rows, n = w_mat.shape
    assert x.shape == (N_DEV * m_rows, m_rows)
    x = x.astype(jnp.float8_e5m2)
    w_mat = w_mat.astype(jnp.float8_e5m2)

    half = n // 2

    def body(x_ref, w_ref, sx_ref, sw_ref, out_ref,
             xg_ref, wg_ref,
             a2a_send_sems, a2a_recv_sems,
             ring_send0, ring_send1, ring_recv0, ring_recv1):
        my = lax.axis_index("i")
        right = lax.rem(my + 1, N_DEV)

        barrier = pltpu.get_barrier_semaphore()
        for k in range(1, N_DEV):
            pl.semaphore_signal(
                barrier, inc=1,
                device_id=(lax.rem(my + k, N_DEV),),
                device_id_type=pl.DeviceIdType.MESH,
            )
        pl.semaphore_wait(barrier, N_DEV - 1)


        a2a = []
        for dj in range(1, N_DEV):
            dst = lax.rem(my + dj, N_DEV)
            r = pltpu.make_async_remote_copy(
                src_ref=x_ref.at[pl.ds(dst * m_rows, m_rows), :],
                dst_ref=xg_ref.at[dj],
                send_sem=a2a_send_sems.at[dj],
                recv_sem=a2a_recv_sems.at[dj],
                device_id=(dst,),
                device_id_type=pl.DeviceIdType.MESH,
            )
            r.start()
            a2a.append(r)

        scale = sx_ref[0] * sw_ref[0]

        def desc(h, s):
            send_sems = ring_send0 if s == 0 else ring_send1
            recv_sems = ring_recv0 if s == 0 else ring_recv1
            src = (w_ref.at[:, pl.ds(s * half, half)] if h == 0
                   else wg_ref.at[h, :, pl.ds(s * half, half)])
            return pltpu.make_async_remote_copy(
                src_ref=src,
                dst_ref=wg_ref.at[h + 1, :, pl.ds(s * half, half)],
                send_sem=send_sems.at[h],
                recv_sem=recv_sems.at[h + 1],
                device_id=(right,),
                device_id_type=pl.DeviceIdType.MESH,
            )

        descs = [[desc(h, 0), desc(h, 1)] for h in range(N_DEV - 1)]
        descs[0][0].start()
        descs[0][1].start()
        out_ref[:, :] = _mm(x_ref[pl.ds(my * m_rows, m_rows), :], w_ref[:, :])

        for h in range(N_DEV - 1):
            descs[h][0].wait_recv()
            if h < N_DEV - 2:
                descs[h + 1][0].start()
                descs[h][1].wait_recv()
                descs[h + 1][1].start()
                a2a[h].wait_recv()
                out_ref[:, :] += _mm(xg_ref[h + 1], wg_ref[h + 1])

        last = N_DEV - 1
        a2a[last - 1].wait_recv()
        acc0 = (out_ref[:, :half]
                + _mm(xg_ref[last], wg_ref[last, :, :half]))
        out_ref[:, :half] = jnp.maximum(acc0 * scale, 0.0)
        descs[last - 1][1].wait_recv()
        acc1 = (out_ref[:, half:]
                + _mm(xg_ref[last], wg_ref[last, :, half:]))
        out_ref[:, half:] = jnp.maximum(acc1 * scale, 0.0)

        for pair in descs:
            pair[0].wait_send()
            pair[1].wait_send()
        for r in a2a:
            r.wait_send()

    return pl.pallas_call(
        body,
        out_shape=jax.ShapeDtypeStruct((m_rows, n), jnp.float32),
        in_specs=[
            pl.BlockSpec(memory_space=pltpu.VMEM),
            pl.BlockSpec(memory_space=pltpu.VMEM),
            pl.BlockSpec(memory_space=pltpu.SMEM),
            pl.BlockSpec(memory_space=pltpu.SMEM),
        ],
        out_specs=pl.BlockSpec(memory_space=pltpu.VMEM),
        scratch_shapes=[
            pltpu.VMEM((N_DEV, m_rows, m_rows), x.dtype),
            pltpu.VMEM((N_DEV, m_rows, n), w_mat.dtype),
            pltpu.SemaphoreType.DMA((N_DEV,)),
            pltpu.SemaphoreType.DMA((N_DEV,)),
            pltpu.SemaphoreType.DMA((N_DEV,)),
            pltpu.SemaphoreType.DMA((N_DEV,)),
            pltpu.SemaphoreType.DMA((N_DEV,)),
            pltpu.SemaphoreType.DMA((N_DEV,)),
        ],
        compiler_params=pltpu.CompilerParams(
            collective_id=0,
            vmem_limit_bytes=56 * 1024 * 1024,
        ),
    )(x, w_mat, scale_x, scale_w)


# device time: 234678 ns/iter; 1.6077x vs baseline; 1.6077x over previous
import jax
import jax.numpy as jnp
from jax import lax
from jax.experimental import pallas as pl
from jax.experimental.pallas import tpu as pltpu

N_DEV = 16


def _perm(p):
    return jnp.where(
        p == 0, 0,
        jnp.where(p <= 4, 4 * (p - 1) + 1,
                  jnp.where(p <= 8, 4 * (8 - p) + 2,
                            jnp.where(p <= 12, 4 * (p - 9) + 3,
                                      4 * (16 - p)))))


def _ringpos(m):
    z = m // 4
    o = m % 4
    return jnp.where(
        o == 0, jnp.where(z == 0, 0, 16 - z),
        jnp.where(o == 1, 1 + z,
                  jnp.where(o == 2, 8 - z, 9 + z)))


def _mm(a, b):
    return lax.dot_general(
        a.astype(jnp.bfloat16),
        b.astype(jnp.bfloat16),
        (((1,), (0,)), ((), ())),
        preferred_element_type=jnp.float32,
    )


def kernel(x, w_mat, scale_x, scale_w):
    m_rows, n = w_mat.shape
    assert x.shape == (N_DEV * m_rows, m_rows)
    x = x.astype(jnp.float8_e5m2)
    w_mat = w_mat.astype(jnp.float8_e5m2)
    half = n // 2

    def body(x_ref, w_ref, sx_ref, sw_ref, out_ref,
             xg_ref, wg_ref,
             a2a_send_sems, a2a_recv_sems,
             fwd_send, fwd_recv, bwd_send, bwd_recv):
        my = lax.axis_index("i")
        rp = _ringpos(my)
        right = _perm(lax.rem(rp + 1, N_DEV))
        left = _perm(lax.rem(rp + N_DEV - 1, N_DEV))

        barrier = pltpu.get_barrier_semaphore()
        for k in range(1, N_DEV):
            pl.semaphore_signal(
                barrier, inc=1,
                device_id=(lax.rem(my + k, N_DEV),),
                device_id_type=pl.DeviceIdType.MESH,
            )
        pl.semaphore_wait(barrier, N_DEV - 1)

        a2a = []
        for dj in range(1, N_DEV):
            mr = _perm(lax.rem(rp + dj, N_DEV))
            r = pltpu.make_async_remote_copy(
                src_ref=x_ref.at[pl.ds(mr * m_rows, m_rows), :],
                dst_ref=xg_ref.at[dj],
                send_sem=a2a_send_sems.at[dj],
                recv_sem=a2a_recv_sems.at[dj],
                device_id=(mr,),
                device_id_type=pl.DeviceIdType.MESH,
            )
            r.start()
            a2a.append(r)

        scale = sx_ref[0] * sw_ref[0]

        def fwd_desc(h):
            src = (w_ref.at[:, pl.ds(0, half)] if h == 0
                   else wg_ref.at[h, :, pl.ds(0, half)])
            return pltpu.make_async_remote_copy(
                src_ref=src,
                dst_ref=wg_ref.at[h + 1, :, pl.ds(0, half)],
                send_sem=fwd_send.at[h],
                recv_sem=fwd_recv.at[h],
                device_id=(right,),
                device_id_type=pl.DeviceIdType.MESH,
            )

        def bwd_desc(h):
            src = (w_ref.at[:, pl.ds(half, half)] if h == 0
                   else wg_ref.at[N_DEV - h, :, pl.ds(half, half)])
            return pltpu.make_async_remote_copy(
                src_ref=src,
                dst_ref=wg_ref.at[N_DEV - 1 - h, :, pl.ds(half, half)],
                send_sem=bwd_send.at[h],
                recv_sem=bwd_recv.at[h],
                device_id=(left,),
                device_id_type=pl.DeviceIdType.MESH,
            )

        fwd = [fwd_desc(h) for h in range(N_DEV - 1)]
        bwd = [bwd_desc(h) for h in range(N_DEV - 1)]
        fwd[0].start()
        bwd[0].start()
        out_ref[:, :] = _mm(x_ref[pl.ds(my * m_rows, m_rows), :], w_ref[:, :])

        for h in range(N_DEV - 1):
            fwd[h].wait_recv()
            if h < N_DEV - 2:
                fwd[h + 1].start()
            bwd[h].wait_recv()
            if h < N_DEV - 2:
                bwd[h + 1].start()
            if h == 7:
                ready = [8]
            elif h > 7:
                ready = [h + 1, N_DEV - 1 - h]
            else:
                ready = []
            for s in ready:
                a2a[s - 1].wait_recv()
                out_ref[:, :] += _mm(xg_ref[s], wg_ref[s])

        out_ref[:, :] = jnp.maximum(out_ref[:, :] * scale, 0.0)

        for r in fwd + bwd + a2a:
            r.wait_send()

    return pl.pallas_call(
        body,
        out_shape=jax.ShapeDtypeStruct((m_rows, n), jnp.float32),
        in_specs=[
            pl.BlockSpec(memory_space=pltpu.VMEM),
            pl.BlockSpec(memory_space=pltpu.VMEM),
            pl.BlockSpec(memory_space=pltpu.SMEM),
            pl.BlockSpec(memory_space=pltpu.SMEM),
        ],
        out_specs=pl.BlockSpec(memory_space=pltpu.VMEM),
        scratch_shapes=[
            pltpu.VMEM((N_DEV, m_rows, m_rows), x.dtype),
            pltpu.VMEM((N_DEV, m_rows, n), w_mat.dtype),
            pltpu.SemaphoreType.DMA((N_DEV,)),
            pltpu.SemaphoreType.DMA((N_DEV,)),
            pltpu.SemaphoreType.DMA((N_DEV,)),
            pltpu.SemaphoreType.DMA((N_DEV,)),
            pltpu.SemaphoreType.DMA((N_DEV,)),
            pltpu.SemaphoreType.DMA((N_DEV,)),
        ],
        compiler_params=pltpu.CompilerParams(
            collective_id=0,
            vmem_limit_bytes=56 * 1024 * 1024,
        ),
    )(x, w_mat, scale_x, scale_w)


# device time: 231486 ns/iter; 1.6299x vs baseline; 1.0138x over previous
import jax
import jax.numpy as jnp
from jax import lax
from jax.experimental import pallas as pl
from jax.experimental.pallas import tpu as pltpu

N_DEV = 16


def _perm(p):
    return jnp.where(
        p == 0, 0,
        jnp.where(p <= 4, 4 * (p - 1) + 1,
                  jnp.where(p <= 8, 4 * (8 - p) + 2,
                            jnp.where(p <= 12, 4 * (p - 9) + 3,
                                      4 * (16 - p)))))


def _ringpos(m):
    z = m // 4
    o = m % 4
    return jnp.where(
        o == 0, jnp.where(z == 0, 0, 16 - z),
        jnp.where(o == 1, 1 + z,
                  jnp.where(o == 2, 8 - z, 9 + z)))


def _mm(a, b):
    return lax.dot_general(
        a.astype(jnp.bfloat16),
        b.astype(jnp.bfloat16),
        (((1,), (0,)), ((), ())),
        preferred_element_type=jnp.float32,
    )


def kernel(x, w_mat, scale_x, scale_w):
    m_rows, n = w_mat.shape
    assert x.shape == (N_DEV * m_rows, m_rows)
    x = x.astype(jnp.float8_e5m2)
    w_mat = w_mat.astype(jnp.float8_e5m2)
    half = n // 2

    def body(x_ref, w_ref, sx_ref, sw_ref, out_ref,
             xg_ref, wg_ref,
             a2a_send_sems, a2a_recv_sems,
             fwd_send, fwd_recv, bwd_send, bwd_recv):
        my = lax.axis_index("i")
        rp = _ringpos(my)
        right = _perm(lax.rem(rp + 1, N_DEV))
        left = _perm(lax.rem(rp + N_DEV - 1, N_DEV))

        barrier = pltpu.get_barrier_semaphore()
        for k in range(1, N_DEV):
            pl.semaphore_signal(
                barrier, inc=1,
                device_id=(lax.rem(my + k, N_DEV),),
                device_id_type=pl.DeviceIdType.MESH,
            )
        pl.semaphore_wait(barrier, N_DEV - 1)

        scale = sx_ref[0] * sw_ref[0]

        def fwd_desc(h):
            src = (w_ref.at[:, pl.ds(0, half)] if h == 0
                   else wg_ref.at[h, :, pl.ds(0, half)])
            return pltpu.make_async_remote_copy(
                src_ref=src,
                dst_ref=wg_ref.at[h + 1, :, pl.ds(0, half)],
                send_sem=fwd_send.at[h],
                recv_sem=fwd_recv.at[h],
                device_id=(right,),
                device_id_type=pl.DeviceIdType.MESH,
            )

        def bwd_desc(h):
            src = (w_ref.at[:, pl.ds(half, half)] if h == 0
                   else wg_ref.at[N_DEV - h, :, pl.ds(half, half)])
            return pltpu.make_async_remote_copy(
                src_ref=src,
                dst_ref=wg_ref.at[N_DEV - 1 - h, :, pl.ds(half, half)],
                send_sem=bwd_send.at[h],
                recv_sem=bwd_recv.at[h],
                device_id=(left,),
                device_id_type=pl.DeviceIdType.MESH,
            )

        fwd = [fwd_desc(h) for h in range(N_DEV - 1)]
        bwd = [bwd_desc(h) for h in range(N_DEV - 1)]
        fwd[0].start()
        bwd[0].start()

        a2a = []
        for dj in range(1, N_DEV):
            mr = _perm(lax.rem(rp + dj, N_DEV))
            r = pltpu.make_async_remote_copy(
                src_ref=x_ref.at[pl.ds(mr * m_rows, m_rows), :],
                dst_ref=xg_ref.at[dj],
                send_sem=a2a_send_sems.at[dj],
                recv_sem=a2a_recv_sems.at[dj],
                device_id=(mr,),
                device_id_type=pl.DeviceIdType.MESH,
            )
            r.start()
            a2a.append(r)

        out_ref[:, :] = _mm(x_ref[pl.ds(my * m_rows, m_rows), :], w_ref[:, :])

        waited = set()

        def need_xg(s):
            if s - 1 not in waited:
                a2a[s - 1].wait_recv()
                waited.add(s - 1)

        for h in range(N_DEV - 1):
            fwd[h].wait_recv()
            if h < N_DEV - 2:
                fwd[h + 1].start()
            bwd[h].wait_recv()
            if h < N_DEV - 2:
                bwd[h + 1].start()
            s_f, s_b = h + 1, N_DEV - 1 - h
            if h < N_DEV - 2:
                need_xg(s_f)
                out_ref[:, :half] += _mm(xg_ref[s_f], wg_ref[s_f, :, :half])
                need_xg(s_b)
                out_ref[:, half:] += _mm(xg_ref[s_b], wg_ref[s_b, :, half:])
            else:
                need_xg(s_f)
                acc0 = (out_ref[:, :half]
                        + _mm(xg_ref[s_f], wg_ref[s_f, :, :half]))
                out_ref[:, :half] = jnp.maximum(acc0 * scale, 0.0)
                need_xg(s_b)
                acc1 = (out_ref[:, half:]
                        + _mm(xg_ref[s_b], wg_ref[s_b, :, half:]))
                out_ref[:, half:] = jnp.maximum(acc1 * scale, 0.0)

        for r in fwd + bwd + a2a:
            r.wait_send()

    return pl.pallas_call(
        body,
        out_shape=jax.ShapeDtypeStruct((m_rows, n), jnp.float32),
        in_specs=[
            pl.BlockSpec(memory_space=pltpu.VMEM),
            pl.BlockSpec(memory_space=pltpu.VMEM),
            pl.BlockSpec(memory_space=pltpu.SMEM),
            pl.BlockSpec(memory_space=pltpu.SMEM),
        ],
        out_specs=pl.BlockSpec(memory_space=pltpu.VMEM),
        scratch_shapes=[
            pltpu.VMEM((N_DEV, m_rows, m_rows), x.dtype),
            pltpu.VMEM((N_DEV, m_rows, n), w_mat.dtype),
            pltpu.SemaphoreType.DMA((N_DEV,)),
            pltpu.SemaphoreType.DMA((N_DEV,)),
            pltpu.SemaphoreType.DMA((N_DEV,)),
            pltpu.SemaphoreType.DMA((N_DEV,)),
            pltpu.SemaphoreType.DMA((N_DEV,)),
            pltpu.SemaphoreType.DMA((N_DEV,)),
        ],
        compiler_params=pltpu.CompilerParams(
            collective_id=0,
            vmem_limit_bytes=56 * 1024 * 1024,
        ),
    )(x, w_mat, scale_x, scale_w)


# device time: 206918 ns/iter; 1.8234x vs baseline; 1.1187x over previous
import jax
import jax.numpy as jnp
from jax import lax
from jax.experimental import pallas as pl
from jax.experimental.pallas import tpu as pltpu

N_DEV = 16


def _perm(p):
    return jnp.where(
        p == 0, 0,
        jnp.where(p <= 4, 4 * (p - 1) + 1,
                  jnp.where(p <= 8, 4 * (8 - p) + 2,
                            jnp.where(p <= 12, 4 * (p - 9) + 3,
                                      4 * (16 - p)))))


def _ringpos(m):
    z = m // 4
    o = m % 4
    return jnp.where(
        o == 0, jnp.where(z == 0, 0, 16 - z),
        jnp.where(o == 1, 1 + z,
                  jnp.where(o == 2, 8 - z, 9 + z)))


def _mm(a, b):
    return lax.dot_general(
        a.astype(jnp.bfloat16),
        b.astype(jnp.bfloat16),
        (((1,), (0,)), ((), ())),
        preferred_element_type=jnp.float32,
    )


def kernel(x, w_mat, scale_x, scale_w):
    m_rows, n = w_mat.shape
    assert x.shape == (N_DEV * m_rows, m_rows)
    x = x.astype(jnp.float8_e5m2)
    w_mat = w_mat.astype(jnp.float8_e5m2)
    half = n // 2

    def body(x_ref, w_ref, sx_ref, sw_ref, out_ref,
             xg_ref, wg_ref,
             a2a_send_sems, a2a_recv_sems,
             fwd_send0, fwd_recv0, fwd_send1, fwd_recv1,
             bwd_send0, bwd_recv0, bwd_send1, bwd_recv1):
        my = lax.axis_index("i")
        rp = _ringpos(my)
        right = _perm(lax.rem(rp + 1, N_DEV))
        left = _perm(lax.rem(rp + N_DEV - 1, N_DEV))

        barrier = pltpu.get_barrier_semaphore()
        for k in range(1, N_DEV):
            pl.semaphore_signal(
                barrier, inc=1,
                device_id=(lax.rem(my + k, N_DEV),),
                device_id_type=pl.DeviceIdType.MESH,
            )
        pl.semaphore_wait(barrier, N_DEV - 1)

        scale = sx_ref[0] * sw_ref[0]

        quarter = half // 2

        def fwd_desc(h, q):
            col = pl.ds(q * quarter, quarter)
            src = (w_ref.at[:, col] if h == 0
                   else wg_ref.at[h, :, col])
            sems = (fwd_send0, fwd_recv0) if q == 0 else (fwd_send1, fwd_recv1)
            return pltpu.make_async_remote_copy(
                src_ref=src,
                dst_ref=wg_ref.at[h + 1, :, col],
                send_sem=sems[0].at[h],
                recv_sem=sems[1].at[h],
                device_id=(right,),
                device_id_type=pl.DeviceIdType.MESH,
            )

        def bwd_desc(h, q):
            col = pl.ds((2 + q) * quarter, quarter)
            src = (w_ref.at[:, col] if h == 0
                   else wg_ref.at[N_DEV - h, :, col])
            sems = (bwd_send0, bwd_recv0) if q == 0 else (bwd_send1, bwd_recv1)
            return pltpu.make_async_remote_copy(
                src_ref=src,
                dst_ref=wg_ref.at[N_DEV - 1 - h, :, col],
                send_sem=sems[0].at[h],
                recv_sem=sems[1].at[h],
                device_id=(left,),
                device_id_type=pl.DeviceIdType.MESH,
            )

        fwd = [[fwd_desc(h, 0), fwd_desc(h, 1)] for h in range(N_DEV - 1)]
        bwd = [[bwd_desc(h, 0), bwd_desc(h, 1)] for h in range(N_DEV - 1)]
        fwd[0][0].start()
        bwd[0][0].start()
        fwd[0][1].start()
        bwd[0][1].start()

        a2a = []
        for dj in range(1, N_DEV):
            mr = _perm(lax.rem(rp + dj, N_DEV))
            r = pltpu.make_async_remote_copy(
                src_ref=x_ref.at[pl.ds(mr * m_rows, m_rows), :],
                dst_ref=xg_ref.at[dj],
                send_sem=a2a_send_sems.at[dj],
                recv_sem=a2a_recv_sems.at[dj],
                device_id=(mr,),
                device_id_type=pl.DeviceIdType.MESH,
            )
            r.start()
            a2a.append(r)

        out_ref[:, :] = _mm(x_ref[pl.ds(my * m_rows, m_rows), :], w_ref[:, :])

        waited = set()

        def need_xg(s):
            if s - 1 not in waited:
                a2a[s - 1].wait_recv()
                waited.add(s - 1)

        for h in range(N_DEV - 1):
            fwd[h][0].wait_recv()
            if h < N_DEV - 2:
                fwd[h + 1][0].start()
            bwd[h][0].wait_recv()
            if h < N_DEV - 2:
                bwd[h + 1][0].start()
            fwd[h][1].wait_recv()
            if h < N_DEV - 2:
                fwd[h + 1][1].start()
            bwd[h][1].wait_recv()
            if h < N_DEV - 2:
                bwd[h + 1][1].start()
            s_f, s_b = h + 1, N_DEV - 1 - h
            if h < N_DEV - 2:
                need_xg(s_f)
                out_ref[:, :half] += _mm(xg_ref[s_f], wg_ref[s_f, :, :half])
                need_xg(s_b)
                out_ref[:, half:] += _mm(xg_ref[s_b], wg_ref[s_b, :, half:])
            else:
                need_xg(s_f)
                acc0 = (out_ref[:, :half]
                        + _mm(xg_ref[s_f], wg_ref[s_f, :, :half]))
                out_ref[:, :half] = jnp.maximum(acc0 * scale, 0.0)
                need_xg(s_b)
                acc1 = (out_ref[:, half:]
                        + _mm(xg_ref[s_b], wg_ref[s_b, :, half:]))
                out_ref[:, half:] = jnp.maximum(acc1 * scale, 0.0)

        for pair in fwd + bwd:
            pair[0].wait_send()
            pair[1].wait_send()
        for r in a2a:
            r.wait_send()

    return pl.pallas_call(
        body,
        out_shape=jax.ShapeDtypeStruct((m_rows, n), jnp.float32),
        in_specs=[
            pl.BlockSpec(memory_space=pltpu.VMEM),
            pl.BlockSpec(memory_space=pltpu.VMEM),
            pl.BlockSpec(memory_space=pltpu.SMEM),
            pl.BlockSpec(memory_space=pltpu.SMEM),
        ],
        out_specs=pl.BlockSpec(memory_space=pltpu.VMEM),
        scratch_shapes=[
            pltpu.VMEM((N_DEV, m_rows, m_rows), x.dtype),
            pltpu.VMEM((N_DEV, m_rows, n), w_mat.dtype),
            pltpu.SemaphoreType.DMA((N_DEV,)),
            pltpu.SemaphoreType.DMA((N_DEV,)),
            pltpu.SemaphoreType.DMA((N_DEV,)),
            pltpu.SemaphoreType.DMA((N_DEV,)),
            pltpu.SemaphoreType.DMA((N_DEV,)),
            pltpu.SemaphoreType.DMA((N_DEV,)),
            pltpu.SemaphoreType.DMA((N_DEV,)),
            pltpu.SemaphoreType.DMA((N_DEV,)),
            pltpu.SemaphoreType.DMA((N_DEV,)),
            pltpu.SemaphoreType.DMA((N_DEV,)),
        ],
        compiler_params=pltpu.CompilerParams(
            collective_id=0,
            vmem_limit_bytes=56 * 1024 * 1024,
        ),
    )(x, w_mat, scale_x, scale_w)


# device time: 203969 ns/iter; 1.8497x vs baseline; 1.0145x over previous
import jax
import jax.numpy as jnp
from jax import lax
from jax.experimental import pallas as pl
from jax.experimental.pallas import tpu as pltpu

N_DEV = 16


def _perm(p):
    return jnp.where(
        p == 0, 0,
        jnp.where(p <= 4, 4 * (p - 1) + 1,
                  jnp.where(p <= 8, 4 * (8 - p) + 2,
                            jnp.where(p <= 12, 4 * (p - 9) + 3,
                                      4 * (16 - p)))))


def _ringpos(m):
    z = m // 4
    o = m % 4
    return jnp.where(
        o == 0, jnp.where(z == 0, 0, 16 - z),
        jnp.where(o == 1, 1 + z,
                  jnp.where(o == 2, 8 - z, 9 + z)))


def _mm(a, b):
    return lax.dot_general(
        a.astype(jnp.bfloat16),
        b.astype(jnp.bfloat16),
        (((1,), (0,)), ((), ())),
        preferred_element_type=jnp.float32,
    )


def kernel(x, w_mat, scale_x, scale_w):
    m_rows, n = w_mat.shape
    assert x.shape == (N_DEV * m_rows, m_rows)
    x = x.astype(jnp.float8_e5m2)
    half = n // 2

    def body(x_ref, w_ref, sx_ref, sw_ref, out_ref,
             xg_ref, wg_ref,
             a2a_send_sems, a2a_recv_sems,
             fwd_send0, fwd_recv0, fwd_send1, fwd_recv1,
             bwd_send0, bwd_recv0, bwd_send1, bwd_recv1):
        my = lax.axis_index("i")
        rp = _ringpos(my)
        right = _perm(lax.rem(rp + 1, N_DEV))
        left = _perm(lax.rem(rp + N_DEV - 1, N_DEV))

        barrier = pltpu.get_barrier_semaphore()
        for k in range(1, N_DEV):
            pl.semaphore_signal(
                barrier, inc=1,
                device_id=(lax.rem(my + k, N_DEV),),
                device_id_type=pl.DeviceIdType.MESH,
            )
        pl.semaphore_wait(barrier, N_DEV - 1)

        scale = sx_ref[0] * sw_ref[0]

        quarter = half // 2

        def fwd_desc(h, q):
            col = pl.ds(q * quarter, quarter)
            src = wg_ref.at[h, :, col]
            sems = (fwd_send0, fwd_recv0) if q == 0 else (fwd_send1, fwd_recv1)
            return pltpu.make_async_remote_copy(
                src_ref=src,
                dst_ref=wg_ref.at[h + 1, :, col],
                send_sem=sems[0].at[h],
                recv_sem=sems[1].at[h],
                device_id=(right,),
                device_id_type=pl.DeviceIdType.MESH,
            )

        def bwd_desc(h, q):
            col = pl.ds((2 + q) * quarter, quarter)
            src = (wg_ref.at[0, :, col] if h == 0
                   else wg_ref.at[N_DEV - h, :, col])
            sems = (bwd_send0, bwd_recv0) if q == 0 else (bwd_send1, bwd_recv1)
            return pltpu.make_async_remote_copy(
                src_ref=src,
                dst_ref=wg_ref.at[N_DEV - 1 - h, :, col],
                send_sem=sems[0].at[h],
                recv_sem=sems[1].at[h],
                device_id=(left,),
                device_id_type=pl.DeviceIdType.MESH,
            )

        fwd = [[fwd_desc(h, 0), fwd_desc(h, 1)] for h in range(N_DEV - 1)]
        bwd = [[bwd_desc(h, 0), bwd_desc(h, 1)] for h in range(N_DEV - 1)]
        for q, r in [(0, fwd[0][0]), (2, bwd[0][0]),
                     (1, fwd[0][1]), (3, bwd[0][1])]:
            col = pl.ds(q * quarter, quarter)
            wg_ref[0, :, col] = w_ref[:, col].astype(jnp.float8_e5m2)
            r.start()

        a2a = []
        for dj in range(1, N_DEV):
            mr = _perm(lax.rem(rp + dj, N_DEV))
            r = pltpu.make_async_remote_copy(
                src_ref=x_ref.at[pl.ds(mr * m_rows, m_rows), :],
                dst_ref=xg_ref.at[dj],
                send_sem=a2a_send_sems.at[dj],
                recv_sem=a2a_recv_sems.at[dj],
                device_id=(mr,),
                device_id_type=pl.DeviceIdType.MESH,
            )
            r.start()
            a2a.append(r)

        out_ref[:, :] = _mm(x_ref[pl.ds(my * m_rows, m_rows), :], w_ref[:, :])

        waited = set()

        def need_xg(s):
            if s - 1 not in waited:
                a2a[s - 1].wait_recv()
                waited.add(s - 1)

        for h in range(N_DEV - 1):
            fwd[h][0].wait_recv()
            if h < N_DEV - 2:
                fwd[h + 1][0].start()
            bwd[h][0].wait_recv()
            if h < N_DEV - 2:
                bwd[h + 1][0].start()
            fwd[h][1].wait_recv()
            if h < N_DEV - 2:
                fwd[h + 1][1].start()
            bwd[h][1].wait_recv()
            if h < N_DEV - 2:
                bwd[h + 1][1].start()
            s_f, s_b = h + 1, N_DEV - 1 - h
            if h < N_DEV - 2:
                need_xg(s_f)
                out_ref[:, :half] += _mm(xg_ref[s_f], wg_ref[s_f, :, :half])
                need_xg(s_b)
                out_ref[:, half:] += _mm(xg_ref[s_b], wg_ref[s_b, :, half:])
            else:
                need_xg(s_f)
                acc0 = (out_ref[:, :half]
                        + _mm(xg_ref[s_f], wg_ref[s_f, :, :half]))
                out_ref[:, :half] = jnp.maximum(acc0 * scale, 0.0)
                need_xg(s_b)
                acc1 = (out_ref[:, half:]
                        + _mm(xg_ref[s_b], wg_ref[s_b, :, half:]))
                out_ref[:, half:] = jnp.maximum(acc1 * scale, 0.0)

        for pair in fwd + bwd:
            pair[0].wait_send()
            pair[1].wait_send()
        for r in a2a:
            r.wait_send()

    return pl.pallas_call(
        body,
        out_shape=jax.ShapeDtypeStruct((m_rows, n), jnp.float32),
        in_specs=[
            pl.BlockSpec(memory_space=pltpu.VMEM),
            pl.BlockSpec(memory_space=pltpu.VMEM),
            pl.BlockSpec(memory_space=pltpu.SMEM),
            pl.BlockSpec(memory_space=pltpu.SMEM),
        ],
        out_specs=pl.BlockSpec(memory_space=pltpu.VMEM),
        scratch_shapes=[
            pltpu.VMEM((N_DEV, m_rows, m_rows), x.dtype),
            pltpu.VMEM((N_DEV, m_rows, n), jnp.float8_e5m2),
            pltpu.SemaphoreType.DMA((N_DEV,)),
            pltpu.SemaphoreType.DMA((N_DEV,)),
            pltpu.SemaphoreType.DMA((N_DEV,)),
            pltpu.SemaphoreType.DMA((N_DEV,)),
            pltpu.SemaphoreType.DMA((N_DEV,)),
            pltpu.SemaphoreType.DMA((N_DEV,)),
            pltpu.SemaphoreType.DMA((N_DEV,)),
            pltpu.SemaphoreType.DMA((N_DEV,)),
            pltpu.SemaphoreType.DMA((N_DEV,)),
            pltpu.SemaphoreType.DMA((N_DEV,)),
        ],
        compiler_params=pltpu.CompilerParams(
            collective_id=0,
            vmem_limit_bytes=56 * 1024 * 1024,
        ),
    )(x, w_mat, scale_x, scale_w)
